# baseline (speedup 1.0000x reference)
import functools

import jax
import jax.numpy as jnp
from jax import lax
from jax.experimental import pallas as pl
from jax.experimental.pallas import tpu as pltpu
from jax.experimental.pallas import tpu_sc as plsc

_K = 512
_L = 16


def _scan_desc(hist, nvregs, carry0, kk):
    iota = lax.iota(jnp.int32, _L)

    def body(t, st):
        carry, found, bkt, cabove, ebin = st
        j = nvregs - 1 - t
        h = hist[pl.ds(j * _L, _L)]
        s = lax.rev(plsc.cumsum(lax.rev(h, (0,))), (0,))
        ge = (carry + s) >= kk
        cnt_ge = jnp.sum(ge.astype(jnp.int32))
        crossed = jnp.logical_and(found == 0, cnt_ge > 0)
        lc = cnt_ge - 1
        sl = jnp.sum(jnp.where(iota == lc, s, 0))
        hl = jnp.sum(jnp.where(iota == lc, h, 0))
        bkt = jnp.where(crossed, j * _L + lc, bkt)
        cabove = jnp.where(crossed, carry + sl - hl, cabove)
        ebin = jnp.where(crossed, hl, ebin)
        found = jnp.where(crossed, jnp.int32(1), found)
        carry = carry + jnp.sum(h)
        return carry, found, bkt, cabove, ebin

    st = (carry0, jnp.int32(0), jnp.int32(0), jnp.int32(0), jnp.int32(0))
    _, _, bkt, cabove, ebin = lax.fori_loop(0, nvregs, body, st)
    return bkt, cabove, ebin


def _zero(hist, nvregs):
    z = jnp.zeros((_L,), jnp.int32)

    def body(j, c):
        hist[pl.ds(j * _L, _L)] = z
        return c

    lax.fori_loop(0, nvregs, body, 0)


def _sc_topk_body(x_hbm, o_hbm, xbuf, kbuf, hist, sem):
    cid = lax.axis_index("c")
    sid = lax.axis_index("s")
    wid = sid * 2 + cid
    nvec = xbuf.shape[0] // _L
    ones = jnp.ones((_L,), jnp.int32)
    top = jnp.uint32(0x80000000)

    def do_row(r_, _c):
        r = wid * 2 + r_
        pltpu.sync_copy(x_hbm.at[r], xbuf)

        _zero(hist, 2048 // _L)

        def pa(i, c):
            v = xbuf[pl.ds(i * _L, _L)]
            u = lax.bitcast_convert_type(v, jnp.uint32)
            key = jnp.where(u >= top, ~u, u | top)
            kbuf[pl.ds(i * _L, _L)] = key
            b = (key >> 21).astype(jnp.int32)
            plsc.addupdate_scatter(hist, [b], ones)
            return c

        lax.fori_loop(0, nvec, pa, 0)
        b1, ca1, _ = _scan_desc(hist, 2048 // _L, jnp.int32(0), _K)

        _zero(hist, 2048 // _L)
        b1u = b1.astype(jnp.uint32)

        def pb(i, c):
            key = kbuf[pl.ds(i * _L, _L)]
            match = (key >> 21) == b1u
            b = ((key >> 10) & jnp.uint32(0x7FF)).astype(jnp.int32)
            plsc.addupdate_scatter(hist, [b], ones, mask=match)
            return c

        lax.fori_loop(0, nvec, pb, 0)
        b2, ca2, _ = _scan_desc(hist, 2048 // _L, ca1, _K)

        _zero(hist, 1024 // _L)
        pref2 = (b1u << 11) | b2.astype(jnp.uint32)

        def pc(i, c):
            key = kbuf[pl.ds(i * _L, _L)]
            match = (key >> 10) == pref2
            b = (key & jnp.uint32(0x3FF)).astype(jnp.int32)
            plsc.addupdate_scatter(hist, [b], ones, mask=match)
            return c

        lax.fori_loop(0, nvec, pc, 0)
        b3, ca3, _ = _scan_desc(hist, 1024 // _L, ca2, _K)

        thr = (pref2 << 10) | b3.astype(jnp.uint32)
        m = jnp.int32(_K) - ca3

        def po(i, eqc):
            key = kbuf[pl.ds(i * _L, _L)]
            gt = key > thr
            eq = key == thr
            ceq = plsc.cumsum(eq.astype(jnp.int32))
            keep = gt | (eq & ((eqc + ceq) <= m))
            v = xbuf[pl.ds(i * _L, _L)]
            xbuf[pl.ds(i * _L, _L)] = jnp.where(keep, v, jnp.float32(0.0))
            return eqc + jnp.sum(eq.astype(jnp.int32))

        lax.fori_loop(0, nvec, po, jnp.int32(0))

        pltpu.sync_copy(xbuf, o_hbm.at[r])
        return _c

    lax.fori_loop(0, 2, do_row, 0)


@jax.jit
def kernel(x):
    b, n = x.shape
    mesh = plsc.VectorSubcoreMesh(
        core_axis_name="c", subcore_axis_name="s", num_cores=2,
        num_subcores=16)
    run = functools.partial(
        pl.kernel,
        out_type=jax.ShapeDtypeStruct((b, n), jnp.float32),
        mesh=mesh,
        compiler_params=pltpu.CompilerParams(needs_layout_passes=False),
        scratch_types=[
            pltpu.VMEM((n,), jnp.float32),
            pltpu.VMEM((n,), jnp.uint32),
            pltpu.VMEM((2048,), jnp.int32),
            pltpu.SemaphoreType.DMA,
        ],
    )(_sc_topk_body)
    return run(x)


# speedup vs baseline: 3.3106x; 3.3106x over previous
import functools

import jax
import jax.numpy as jnp
from jax import lax
from jax.experimental import pallas as pl
from jax.experimental.pallas import tpu as pltpu
from jax.experimental.pallas import tpu_sc as plsc

_K = 512
_L = 16


def _scan_desc(hist, nvregs, carry0, kk):
    iota = lax.iota(jnp.int32, _L)

    def body(t, st):
        carry, found, bkt, cabove, ebin = st
        j = nvregs - 1 - t
        h = hist[pl.ds(j * _L, _L)]
        s = lax.rev(plsc.cumsum(lax.rev(h, (0,))), (0,))
        ge = (carry + s) >= kk
        cnt_ge = jnp.sum(ge.astype(jnp.int32))
        crossed = jnp.logical_and(found == 0, cnt_ge > 0)
        lc = cnt_ge - 1
        sl = jnp.sum(jnp.where(iota == lc, s, 0))
        hl = jnp.sum(jnp.where(iota == lc, h, 0))
        bkt = jnp.where(crossed, j * _L + lc, bkt)
        cabove = jnp.where(crossed, carry + sl - hl, cabove)
        ebin = jnp.where(crossed, hl, ebin)
        found = jnp.where(crossed, jnp.int32(1), found)
        carry = carry + jnp.sum(h)
        return carry, found, bkt, cabove, ebin

    st = (carry0, jnp.int32(0), jnp.int32(0), jnp.int32(0), jnp.int32(0))
    _, _, bkt, cabove, ebin = lax.fori_loop(0, nvregs, body, st)
    return bkt, cabove, ebin


def _zero(hist, nvregs):
    z = jnp.zeros((_L,), jnp.int32)

    @functools.partial(plsc.parallel_loop, 0, nvregs, unroll=8)
    def body(j):
        hist[pl.ds(j * _L, _L)] = z


def _sc_topk_body(x_hbm, o_hbm, xbuf, kbuf, hist, sem):
    cid = lax.axis_index("c")
    sid = lax.axis_index("s")
    wid = sid * 2 + cid
    nvec = xbuf.shape[0] // _L
    ones = jnp.ones((_L,), jnp.int32)
    top = jnp.uint32(0x80000000)

    def do_row(r_, _c):
        r = wid * 2 + r_
        pltpu.sync_copy(x_hbm.at[r], xbuf)

        _zero(hist, 2048 // _L)

        @functools.partial(plsc.parallel_loop, 0, nvec, unroll=8)
        def pa(i):
            v = xbuf[pl.ds(i * _L, _L)]
            u = lax.bitcast_convert_type(v, jnp.uint32)
            key = jnp.where(u >= top, ~u, u | top)
            kbuf[pl.ds(i * _L, _L)] = key
            b = (key >> 21).astype(jnp.int32)
            plsc.addupdate_scatter(hist, [b], ones)
        b1, ca1, _ = _scan_desc(hist, 2048 // _L, jnp.int32(0), _K)

        _zero(hist, 2048 // _L)
        b1u = b1.astype(jnp.uint32)

        @functools.partial(plsc.parallel_loop, 0, nvec, unroll=8)
        def pb(i):
            key = kbuf[pl.ds(i * _L, _L)]
            match = (key >> 21) == b1u
            b = ((key >> 10) & jnp.uint32(0x7FF)).astype(jnp.int32)
            plsc.addupdate_scatter(hist, [b], ones, mask=match)
        b2, ca2, _ = _scan_desc(hist, 2048 // _L, ca1, _K)

        _zero(hist, 1024 // _L)
        pref2 = (b1u << 11) | b2.astype(jnp.uint32)

        @functools.partial(plsc.parallel_loop, 0, nvec, unroll=8)
        def pc(i):
            key = kbuf[pl.ds(i * _L, _L)]
            match = (key >> 10) == pref2
            b = (key & jnp.uint32(0x3FF)).astype(jnp.int32)
            plsc.addupdate_scatter(hist, [b], ones, mask=match)

        b3, ca3, ebin = _scan_desc(hist, 1024 // _L, ca2, _K)

        thr = (pref2 << 10) | b3.astype(jnp.uint32)
        m = jnp.int32(_K) - ca3

        @pl.when(m == ebin)
        def _simple():
            @functools.partial(plsc.parallel_loop, 0, nvec, unroll=8)
            def po(i):
                key = kbuf[pl.ds(i * _L, _L)]
                keep = key >= thr
                v = xbuf[pl.ds(i * _L, _L)]
                xbuf[pl.ds(i * _L, _L)] = jnp.where(keep, v, jnp.float32(0.0))

        @pl.when(m != ebin)
        def _ties():
            def po(i, eqc):
                key = kbuf[pl.ds(i * _L, _L)]
                gt = key > thr
                eq = key == thr
                ceq = plsc.cumsum(eq.astype(jnp.int32))
                keep = gt | (eq & ((eqc + ceq) <= m))
                v = xbuf[pl.ds(i * _L, _L)]
                xbuf[pl.ds(i * _L, _L)] = jnp.where(keep, v, jnp.float32(0.0))
                return eqc + jnp.sum(eq.astype(jnp.int32))

            lax.fori_loop(0, nvec, po, jnp.int32(0))

        pltpu.sync_copy(xbuf, o_hbm.at[r])
        return _c

    lax.fori_loop(0, 2, do_row, 0)


@jax.jit
def kernel(x):
    b, n = x.shape
    mesh = plsc.VectorSubcoreMesh(
        core_axis_name="c", subcore_axis_name="s", num_cores=2,
        num_subcores=16)
    run = functools.partial(
        pl.kernel,
        out_type=jax.ShapeDtypeStruct((b, n), jnp.float32),
        mesh=mesh,
        compiler_params=pltpu.CompilerParams(needs_layout_passes=False),
        scratch_types=[
            pltpu.VMEM((n,), jnp.float32),
            pltpu.VMEM((n,), jnp.uint32),
            pltpu.VMEM((2048,), jnp.int32),
            pltpu.SemaphoreType.DMA,
        ],
    )(_sc_topk_body)
    return run(x)
